# baseline (speedup 1.0000x reference)
import functools

import jax
import jax.numpy as jnp
from jax import lax
from jax.experimental import pallas as pl
from jax.experimental.pallas import tpu as pltpu
from jax.experimental.pallas import tpu_sc as plsc

B, K, P, C, A = 16, 900, 900, 92, 117
NCLS = 80
HALF = P // 2
CHUNKS = (HALF + 15) // 16
PAD = CHUNKS * 16


def _score_label_body(logits_ref, score_ref, label_ref):
    x = logits_ref[0]
    lane = lax.broadcasted_iota(jnp.int32, (K, C), 1)
    neg = jnp.float32(-jnp.inf)
    x80 = jnp.where(lane < NCLS, x, neg)
    m80 = jnp.max(x80, axis=1, keepdims=True)
    lab = jnp.min(jnp.where(x80 == m80, lane, C), axis=1, keepdims=True)
    x81 = jnp.where(lane < NCLS + 1, x, neg)
    m81 = jnp.max(x81, axis=1, keepdims=True)
    se = jnp.sum(jnp.exp(x81 - m81), axis=1, keepdims=True)
    score_ref[0] = jnp.exp(m80 - m81) / se
    label_ref[0] = lab


_score_label = pl.pallas_call(
    _score_label_body,
    grid=(B,),
    in_specs=[pl.BlockSpec((1, K, C), lambda b: (b, 0, 0))],
    out_specs=[
        pl.BlockSpec((1, K, 1), lambda b: (b, 0, 0)),
        pl.BlockSpec((1, K, 1), lambda b: (b, 0, 0)),
    ],
    out_shape=[
        jax.ShapeDtypeStruct((B, K, 1), jnp.float32),
        jax.ShapeDtypeStruct((B, K, 1), jnp.int32),
    ],
)


_mesh = plsc.VectorSubcoreMesh(core_axis_name="c", subcore_axis_name="s")


@functools.partial(
    pl.kernel,
    out_type=(
        jax.ShapeDtypeStruct((B, 4, HALF), jnp.int32),
        jax.ShapeDtypeStruct((B, 4, HALF, 4), jnp.float32),
        jax.ShapeDtypeStruct((B, 2, HALF), jnp.float32),
    ),
    mesh=_mesh,
    compiler_params=pltpu.CompilerParams(
        needs_layout_passes=False, use_tc_tiling_on_sc=False
    ),
    scratch_types=[
        pltpu.VMEM((PAD,), jnp.int32),
        pltpu.VMEM((PAD,), jnp.int32),
        pltpu.VMEM((K,), jnp.float32),
        pltpu.VMEM((K,), jnp.int32),
        pltpu.VMEM((4 * K,), jnp.float32),
        pltpu.VMEM((2, 16), jnp.float32),
        pltpu.VMEM((PAD,), jnp.int32),
        pltpu.VMEM((PAD,), jnp.int32),
        pltpu.VMEM((PAD,), jnp.float32),
        pltpu.VMEM((PAD, 4), jnp.float32),
        pltpu.VMEM((PAD, 4), jnp.float32),
    ],
)
def _pair_gather(hidx, oidx, score, label, pbox, scale,
                 lab_out, box_out, osc_out,
                 hbuf, obuf, stab, ltab, btab, scl,
                 lab_o, zer, sco_o, bo_h, bo_o):
    b = lax.axis_index("s")
    half = lax.axis_index("c")

    pltpu.sync_copy(hidx.at[b, half], hbuf.at[pl.ds(0, HALF)])
    pltpu.sync_copy(oidx.at[b, half], obuf.at[pl.ds(0, HALF)])
    pltpu.sync_copy(score.at[b], stab)
    pltpu.sync_copy(label.at[b], ltab)
    pltpu.sync_copy(pbox.at[b], btab)
    pltpu.sync_copy(scale.at[b], scl)

    sw = scl[0]
    sh = scl[1]
    iota = lax.iota(jnp.int32, 16)
    zeros16 = jnp.zeros((16,), jnp.int32)

    for i in range(CHUNKS):
        pos = i * 16
        hv = jnp.minimum(jnp.maximum(hbuf[pl.ds(pos, 16)], 0), K - 1)
        ov = jnp.minimum(jnp.maximum(obuf[pl.ds(pos, 16)], 0), K - 1)

        sco_o[pl.ds(pos, 16)] = plsc.load_gather(stab, [ov])
        lab_o[pl.ds(pos, 16)] = plsc.load_gather(ltab, [ov])
        zer[pl.ds(pos, 16)] = zeros16

        rows = iota + pos
        for idxv, bo in ((hv, bo_h), (ov, bo_o)):
            base = idxv * 4
            cx = plsc.load_gather(btab, [base])
            cy = plsc.load_gather(btab, [base + 1])
            hw = plsc.load_gather(btab, [base + 2]) * 0.5
            hh = plsc.load_gather(btab, [base + 3]) * 0.5
            plsc.store_scatter(bo, [rows, zeros16], (cx - hw) * sw)
            plsc.store_scatter(bo, [rows, zeros16 + 1], (cy - hh) * sh)
            plsc.store_scatter(bo, [rows, zeros16 + 2], (cx + hw) * sw)
            plsc.store_scatter(bo, [rows, zeros16 + 3], (cy + hh) * sh)

    pltpu.sync_copy(zer.at[pl.ds(0, HALF)], lab_out.at[b, half])
    pltpu.sync_copy(lab_o.at[pl.ds(0, HALF)], lab_out.at[b, 2 + half])
    pltpu.sync_copy(sco_o.at[pl.ds(0, HALF)], osc_out.at[b, half])
    pltpu.sync_copy(bo_h.at[pl.ds(0, HALF)], box_out.at[b, half])
    pltpu.sync_copy(bo_o.at[pl.ds(0, HALF)], box_out.at[b, 2 + half])


def _verb_body(act_ref, osc_ref, out_ref):
    a = act_ref[0]
    s = osc_ref[0]
    out_ref[0] = s / (1.0 + jnp.exp(-a))


_verb = pl.pallas_call(
    _verb_body,
    grid=(B,),
    in_specs=[
        pl.BlockSpec((1, P, A), lambda b: (b, 0, 0)),
        pl.BlockSpec((1, P, 1), lambda b: (b, 0, 0)),
    ],
    out_specs=pl.BlockSpec((1, P, A), lambda b: (b, 0, 0)),
    out_shape=jax.ShapeDtypeStruct((B, P, A), jnp.float32),
)


def kernel(pred_logits, pred_boxes, pred_actions, pred_rel_pairs, target_sizes):
    score3, label3 = _score_label(pred_logits)

    hidx = pred_rel_pairs[..., 0].reshape(B, 2, HALF)
    oidx = pred_rel_pairs[..., 1].reshape(B, 2, HALF)
    img_w = target_sizes[:, 1].astype(jnp.float32)
    img_h = target_sizes[:, 0].astype(jnp.float32)
    scale = jnp.broadcast_to(
        jnp.stack([img_w, img_h], axis=1)[:, :, None], (B, 2, 16)
    )

    lab4, box4, osc = _pair_gather(
        hidx, oidx,
        score3.reshape(B, K), label3.reshape(B, K),
        pred_boxes.reshape(B, 4 * K), scale,
    )

    verb = _verb(pred_actions, osc.reshape(B, P, 1))
    return lab4.reshape(B, 2 * P), box4.reshape(B, 2 * P, 4), verb


# speedup vs baseline: 1.3090x; 1.3090x over previous
import functools

import jax
import jax.numpy as jnp
from jax import lax
from jax.experimental import pallas as pl
from jax.experimental.pallas import tpu as pltpu
from jax.experimental.pallas import tpu_sc as plsc

B, K, P, C, A = 16, 900, 900, 92, 117
NCLS = 80
KP = 904
HW = 456
PW = 2 * HW
CHUNKS = 29
BB = 8
AB = 13


def _score_label_body(lgt_ref, score_ref, label_ref):
    x = lgt_ref[...]
    cl = lax.broadcasted_iota(jnp.int32, (C, BB, K), 0)
    neg = jnp.float32(-jnp.inf)
    x80 = jnp.where(cl < NCLS, x, neg)
    m80 = jnp.max(x80, axis=0, keepdims=True)
    lab = jnp.min(jnp.where(x80 == m80, cl, C), axis=0)
    x81 = jnp.where(cl < NCLS + 1, x, neg)
    m81 = jnp.max(x81, axis=0, keepdims=True)
    se = jnp.sum(jnp.exp(x81 - m81), axis=0)
    score_ref[:, pl.ds(0, K)] = jnp.exp(m80[0] - m81[0]) / se
    label_ref[:, pl.ds(0, K)] = lab


_score_label = pl.pallas_call(
    _score_label_body,
    grid=(B // BB,),
    in_specs=[pl.BlockSpec((C, BB, K), lambda i: (0, i, 0))],
    out_specs=[
        pl.BlockSpec((BB, KP), lambda i: (i, 0)),
        pl.BlockSpec((BB, KP), lambda i: (i, 0)),
    ],
    out_shape=[
        jax.ShapeDtypeStruct((B, KP), jnp.float32),
        jax.ShapeDtypeStruct((B, KP), jnp.int32),
    ],
)


_mesh = plsc.VectorSubcoreMesh(core_axis_name="c", subcore_axis_name="s")


@functools.partial(
    pl.kernel,
    out_type=(
        jax.ShapeDtypeStruct((B, 2, PW), jnp.int32),
        jax.ShapeDtypeStruct((B, 4, 2, PW), jnp.float32),
        jax.ShapeDtypeStruct((B, PW), jnp.float32),
    ),
    mesh=_mesh,
    compiler_params=pltpu.CompilerParams(
        needs_layout_passes=False, use_tc_tiling_on_sc=False
    ),
    scratch_types=[
        pltpu.VMEM((920,), jnp.int32),
        pltpu.VMEM((920,), jnp.int32),
        pltpu.VMEM((KP,), jnp.float32),
        pltpu.VMEM((KP,), jnp.int32),
        pltpu.VMEM((4, KP), jnp.float32),
        pltpu.VMEM((2, 16), jnp.int32),
        pltpu.VMEM((464,), jnp.int32),
        pltpu.VMEM((464,), jnp.int32),
        pltpu.VMEM((464,), jnp.float32),
        pltpu.VMEM((8, 464), jnp.float32),
    ],
)
def _pair_gather(hido, score, label, pbox, tsizes,
                 lab_out, box_out, osc_out,
                 hbuf, obuf, stab, ltab, btab, tsb,
                 lab_o, zer, sco_o, bo):
    b = lax.axis_index("s")
    half = lax.axis_index("c")
    start = half * HW

    pltpu.sync_copy(hido.at[b, 0], hbuf.at[pl.ds(0, KP)])
    pltpu.sync_copy(hido.at[b, 1], obuf.at[pl.ds(0, KP)])
    pltpu.sync_copy(score.at[b], stab)
    pltpu.sync_copy(label.at[b], ltab)
    pltpu.sync_copy(pbox.at[b], btab)
    pltpu.sync_copy(tsizes, tsb)

    bsplat = jnp.zeros((16,), jnp.int32) + b
    sh = plsc.load_gather(tsb, [jnp.zeros((16,), jnp.int32), bsplat])
    sw = plsc.load_gather(tsb, [jnp.zeros((16,), jnp.int32) + 1, bsplat])
    sh = sh.astype(jnp.float32)
    sw = sw.astype(jnp.float32)

    iota = lax.iota(jnp.int32, 16)
    zeros16 = jnp.zeros((16,), jnp.int32)

    for i in range(CHUNKS):
        pos = i * 16
        hv = jnp.minimum(jnp.maximum(hbuf[pl.ds(start + pos, 16)], 0), K - 1)
        ov = jnp.minimum(jnp.maximum(obuf[pl.ds(start + pos, 16)], 0), K - 1)

        sco_o[pl.ds(pos, 16)] = plsc.load_gather(stab, [ov])
        lab_o[pl.ds(pos, 16)] = plsc.load_gather(ltab, [ov])
        zer[pl.ds(pos, 16)] = zeros16

        for t, idxv in ((0, hv), (1, ov)):
            cx = plsc.load_gather(btab, [zeros16, idxv])
            cy = plsc.load_gather(btab, [zeros16 + 1, idxv])
            hw2 = plsc.load_gather(btab, [zeros16 + 2, idxv]) * 0.5
            hh2 = plsc.load_gather(btab, [zeros16 + 3, idxv]) * 0.5
            bo[4 * t + 0, pl.ds(pos, 16)] = (cx - hw2) * sw
            bo[4 * t + 1, pl.ds(pos, 16)] = (cy - hh2) * sh
            bo[4 * t + 2, pl.ds(pos, 16)] = (cx + hw2) * sw
            bo[4 * t + 3, pl.ds(pos, 16)] = (cy + hh2) * sh

    pltpu.sync_copy(zer.at[pl.ds(0, HW)], lab_out.at[b, 0, pl.ds(start, HW)])
    pltpu.sync_copy(lab_o.at[pl.ds(0, HW)], lab_out.at[b, 1, pl.ds(start, HW)])
    pltpu.sync_copy(sco_o.at[pl.ds(0, HW)], osc_out.at[b, pl.ds(start, HW)])
    for t in range(2):
        for c in range(4):
            pltpu.sync_copy(
                bo.at[4 * t + c, pl.ds(0, HW)],
                box_out.at[b, c, t, pl.ds(start, HW)],
            )


def _verb_body(act_ref, osc_ref, out_ref):
    a = act_ref[...]
    s = osc_ref[:, pl.ds(0, P)]
    out_ref[...] = s[None] / (1.0 + jnp.exp(-a))


_verb = pl.pallas_call(
    _verb_body,
    grid=(A // AB,),
    in_specs=[
        pl.BlockSpec((AB, B, P), lambda i: (i, 0, 0)),
        pl.BlockSpec((B, PW), lambda i: (0, 0)),
    ],
    out_specs=pl.BlockSpec((AB, B, P), lambda i: (i, 0, 0)),
    out_shape=jax.ShapeDtypeStruct((A, B, P), jnp.float32),
)


def kernel(pred_logits, pred_boxes, pred_actions, pred_rel_pairs, target_sizes):
    lgt = jnp.transpose(pred_logits, (2, 0, 1))
    act = jnp.transpose(pred_actions, (2, 0, 1))
    hido = jnp.pad(
        jnp.transpose(pred_rel_pairs, (0, 2, 1)), ((0, 0), (0, 0), (0, KP - P))
    )
    pbox = jnp.pad(
        jnp.transpose(pred_boxes, (0, 2, 1)), ((0, 0), (0, 0), (0, KP - K))
    )
    ts = jnp.transpose(target_sizes, (1, 0))

    score, label = _score_label(lgt)

    lab2, box4, osc = _pair_gather(hido, score, label, pbox, ts)

    verb_t = _verb(act, osc)

    labels = lab2[:, :, :P].reshape(B, 2 * P)
    b_out = jnp.transpose(box4[:, :, :, :P].reshape(B, 4, 2 * P), (0, 2, 1))
    verb = jnp.transpose(verb_t, (1, 2, 0))
    return labels, b_out, verb


# speedup vs baseline: 2.1858x; 1.6698x over previous
import functools

import jax
import jax.numpy as jnp
from jax import lax
from jax.experimental import pallas as pl
from jax.experimental.pallas import tpu as pltpu
from jax.experimental.pallas import tpu_sc as plsc

B, K, P, C, A = 16, 900, 900, 92, 117
NCLS = 80
KP = 904
HW = 456
PW = 2 * HW
CHUNKS = 29
BB = 8
AB = 13


def _score_label_body(lgt_ref, score_ref, label_ref):
    x = lgt_ref[...]
    cl = lax.broadcasted_iota(jnp.int32, (C, BB, K), 0)
    neg = jnp.float32(-jnp.inf)
    x80 = jnp.where(cl < NCLS, x, neg)
    m80 = jnp.max(x80, axis=0, keepdims=True)
    lab = jnp.min(jnp.where(x80 == m80, cl, C), axis=0)
    x81 = jnp.where(cl < NCLS + 1, x, neg)
    m81 = jnp.max(x81, axis=0, keepdims=True)
    se = jnp.sum(jnp.exp(x81 - m81), axis=0)
    score_ref[:, pl.ds(0, K)] = jnp.exp(m80[0] - m81[0]) / se
    label_ref[:, pl.ds(0, K)] = lab


_score_label = pl.pallas_call(
    _score_label_body,
    grid=(B // BB,),
    in_specs=[pl.BlockSpec((C, BB, K), lambda i: (0, i, 0))],
    out_specs=[
        pl.BlockSpec((BB, KP), lambda i: (i, 0)),
        pl.BlockSpec((BB, KP), lambda i: (i, 0)),
    ],
    out_shape=[
        jax.ShapeDtypeStruct((B, KP), jnp.float32),
        jax.ShapeDtypeStruct((B, KP), jnp.int32),
    ],
)


_mesh = plsc.VectorSubcoreMesh(core_axis_name="c", subcore_axis_name="s")


@functools.partial(
    pl.kernel,
    out_type=(
        jax.ShapeDtypeStruct((B, 2, PW), jnp.int32),
        jax.ShapeDtypeStruct((B, 4, 2, PW), jnp.float32),
        jax.ShapeDtypeStruct((B, PW), jnp.float32),
    ),
    mesh=_mesh,
    compiler_params=pltpu.CompilerParams(
        needs_layout_passes=False, use_tc_tiling_on_sc=False
    ),
    scratch_types=[
        pltpu.VMEM((920,), jnp.int32),
        pltpu.VMEM((920,), jnp.int32),
        pltpu.VMEM((KP,), jnp.float32),
        pltpu.VMEM((KP,), jnp.int32),
        pltpu.VMEM((4, KP), jnp.float32),
        pltpu.VMEM((2, 16), jnp.int32),
        pltpu.VMEM((464,), jnp.int32),
        pltpu.VMEM((464,), jnp.int32),
        pltpu.VMEM((464,), jnp.float32),
        pltpu.VMEM((8, 464), jnp.float32),
    ],
)
def _pair_gather(hido, score, label, pbox, tsizes,
                 lab_out, box_out, osc_out,
                 hbuf, obuf, stab, ltab, btab, tsb,
                 lab_o, zer, sco_o, bo):
    b = lax.axis_index("s")
    half = lax.axis_index("c")
    start = half * HW

    pltpu.sync_copy(hido.at[b, 0], hbuf.at[pl.ds(0, KP)])
    pltpu.sync_copy(hido.at[b, 1], obuf.at[pl.ds(0, KP)])
    pltpu.sync_copy(score.at[b], stab)
    pltpu.sync_copy(label.at[b], ltab)
    pltpu.sync_copy(pbox.at[b], btab)
    pltpu.sync_copy(tsizes, tsb)

    bsplat = jnp.zeros((16,), jnp.int32) + b
    sh = plsc.load_gather(tsb, [jnp.zeros((16,), jnp.int32), bsplat])
    sw = plsc.load_gather(tsb, [jnp.zeros((16,), jnp.int32) + 1, bsplat])
    sh = sh.astype(jnp.float32)
    sw = sw.astype(jnp.float32)

    iota = lax.iota(jnp.int32, 16)
    zeros16 = jnp.zeros((16,), jnp.int32)

    for i in range(CHUNKS):
        pos = i * 16
        hv = jnp.minimum(jnp.maximum(hbuf[pl.ds(start + pos, 16)], 0), K - 1)
        ov = jnp.minimum(jnp.maximum(obuf[pl.ds(start + pos, 16)], 0), K - 1)

        sco_o[pl.ds(pos, 16)] = plsc.load_gather(stab, [ov])
        lab_o[pl.ds(pos, 16)] = plsc.load_gather(ltab, [ov])
        zer[pl.ds(pos, 16)] = zeros16

        for t, idxv in ((0, hv), (1, ov)):
            cx = plsc.load_gather(btab, [zeros16, idxv])
            cy = plsc.load_gather(btab, [zeros16 + 1, idxv])
            hw2 = plsc.load_gather(btab, [zeros16 + 2, idxv]) * 0.5
            hh2 = plsc.load_gather(btab, [zeros16 + 3, idxv]) * 0.5
            bo[4 * t + 0, pl.ds(pos, 16)] = (cx - hw2) * sw
            bo[4 * t + 1, pl.ds(pos, 16)] = (cy - hh2) * sh
            bo[4 * t + 2, pl.ds(pos, 16)] = (cx + hw2) * sw
            bo[4 * t + 3, pl.ds(pos, 16)] = (cy + hh2) * sh

    pltpu.sync_copy(zer.at[pl.ds(0, HW)], lab_out.at[b, 0, pl.ds(start, HW)])
    pltpu.sync_copy(lab_o.at[pl.ds(0, HW)], lab_out.at[b, 1, pl.ds(start, HW)])
    pltpu.sync_copy(sco_o.at[pl.ds(0, HW)], osc_out.at[b, pl.ds(start, HW)])
    for t in range(2):
        for c in range(4):
            pltpu.sync_copy(
                bo.at[4 * t + c, pl.ds(0, HW)],
                box_out.at[b, c, t, pl.ds(start, HW)],
            )


PT = 60


def _verb_body(act_ref, osc_ref, out_ref):
    sig = 1.0 / (1.0 + jnp.exp(-act_ref[...]))
    s = osc_ref[0]
    for j in range(PT):
        out_ref[j] = sig[j] * jnp.broadcast_to(s[:, j : j + 1], (B, A))


_verb = pl.pallas_call(
    _verb_body,
    grid=(P // PT,),
    in_specs=[
        pl.BlockSpec((PT, B, A), lambda i: (i, 0, 0)),
        pl.BlockSpec((1, B, PT), lambda i: (i, 0, 0)),
    ],
    out_specs=pl.BlockSpec((PT, B, A), lambda i: (i, 0, 0)),
    out_shape=jax.ShapeDtypeStruct((P, B, A), jnp.float32),
)


def kernel(pred_logits, pred_boxes, pred_actions, pred_rel_pairs, target_sizes):
    lgt = jnp.transpose(pred_logits, (2, 0, 1))
    act = jnp.transpose(pred_actions, (1, 0, 2))
    hido = jnp.pad(
        jnp.transpose(pred_rel_pairs, (0, 2, 1)), ((0, 0), (0, 0), (0, KP - P))
    )
    pbox = jnp.pad(
        jnp.transpose(pred_boxes, (0, 2, 1)), ((0, 0), (0, 0), (0, KP - K))
    )
    ts = jnp.transpose(target_sizes, (1, 0))

    score, label = _score_label(lgt)

    lab2, box4, osc = _pair_gather(hido, score, label, pbox, ts)

    osc_r = jnp.transpose(osc[:, :P].reshape(B, P // PT, PT), (1, 0, 2))
    verb_t = _verb(act, osc_r)

    labels = lab2[:, :, :P].reshape(B, 2 * P)
    b_out = jnp.transpose(box4[:, :, :, :P].reshape(B, 4, 2 * P), (0, 2, 1))
    verb = jnp.transpose(verb_t, (1, 0, 2))
    return labels, b_out, verb
